# baseline (speedup 1.0000x reference)
import functools

import jax
import jax.numpy as jnp
from jax import lax
from jax.experimental import pallas as pl
from jax.experimental.pallas import tpu as pltpu
from jax.experimental.pallas import tpu_sc as plsc

B = 16384
D = 32
NC = 2
NS = 16
NW = NC * NS
BPW = B // NW


def _sc_body(uidx_hbm, iidx_hbm, uemb_hbm, iemb_hbm, ub_hbm, ib_hbm,
             partials_hbm, bias_hbm,
             uidx_v, iidx_v, urows_v, irows_v, ubv_v, ibv_v, acc_v, outb_v,
             sem_u, sem_i, sem_ub, sem_ib):
    wid = lax.axis_index("s") * NC + lax.axis_index("c")
    base = wid * BPW

    pltpu.sync_copy(uidx_hbm.at[pl.ds(base, BPW)], uidx_v)
    pltpu.sync_copy(iidx_hbm.at[pl.ds(base, BPW)], iidx_v)

    cu = pltpu.async_copy(uemb_hbm.at[uidx_v], urows_v, sem_u)
    ci = pltpu.async_copy(iemb_hbm.at[iidx_v], irows_v, sem_i)
    cub = pltpu.async_copy(ub_hbm.at[uidx_v], ubv_v, sem_ub)
    cib = pltpu.async_copy(ib_hbm.at[iidx_v], ibv_v, sem_ib)
    cu.wait()
    ci.wait()

    def dot_body(i, acc):
        u0 = urows_v[i, pl.ds(0, 16)]
        v0 = irows_v[i, pl.ds(0, 16)]
        u1 = urows_v[i, pl.ds(16, 16)]
        v1 = irows_v[i, pl.ds(16, 16)]
        return acc + u0 * v0 + u1 * v1

    acc = lax.fori_loop(0, BPW, dot_body, jnp.zeros((16,), jnp.float32),
                        unroll=4)
    acc_v[...] = acc
    pltpu.sync_copy(acc_v, partials_hbm.at[wid])

    cub.wait()
    cib.wait()
    lanes = lax.iota(jnp.int32, 16)
    zeros16 = jnp.zeros((16,), jnp.int32)
    for j in range(BPW // 16):
        ug = plsc.load_gather(ubv_v, [lanes + j * 16, zeros16])
        ig = plsc.load_gather(ibv_v, [lanes + j * 16, zeros16])
        outb_v[pl.ds(j * 16, 16)] = ug + ig
    pltpu.sync_copy(outb_v, bias_hbm.at[pl.ds(base, BPW)])


@functools.cache
def _make_sc_call():
    return pl.kernel(
        _sc_body,
        out_type=[
            jax.ShapeDtypeStruct((NW, 16), jnp.float32),
            jax.ShapeDtypeStruct((B,), jnp.float32),
        ],
        mesh=plsc.VectorSubcoreMesh(
            core_axis_name="c", subcore_axis_name="s"),
        compiler_params=pltpu.CompilerParams(
            use_tc_tiling_on_sc=False, needs_layout_passes=False),
        scratch_types=[
            pltpu.VMEM((BPW,), jnp.int32),
            pltpu.VMEM((BPW,), jnp.int32),
            pltpu.VMEM((BPW, D), jnp.float32),
            pltpu.VMEM((BPW, D), jnp.float32),
            pltpu.VMEM((BPW, 1), jnp.float32),
            pltpu.VMEM((BPW, 1), jnp.float32),
            pltpu.VMEM((16,), jnp.float32),
            pltpu.VMEM((BPW,), jnp.float32),
            pltpu.SemaphoreType.DMA,
            pltpu.SemaphoreType.DMA,
            pltpu.SemaphoreType.DMA,
            pltpu.SemaphoreType.DMA,
        ],
    )


def _tc_body(bias_ref, partials_ref, out_ref):
    total = jnp.sum(partials_ref[...])
    out_ref[...] = bias_ref[...] + total


_tc_call = pl.pallas_call(
    _tc_body,
    out_shape=jax.ShapeDtypeStruct((128, 128), jnp.float32),
)


def kernel(user_index, item_index, user_emb, item_emb, ub, ib):
    partials, bias = _make_sc_call()(
        user_index.astype(jnp.int32), item_index.astype(jnp.int32),
        user_emb, item_emb, ub, ib)
    out2d = _tc_call(bias.reshape(128, 128), partials)
    return out2d.reshape(B)


# speedup vs baseline: 2.7890x; 2.7890x over previous
import functools

import jax
import jax.numpy as jnp
from jax import lax
from jax.experimental import pallas as pl
from jax.experimental.pallas import tpu as pltpu
from jax.experimental.pallas import tpu_sc as plsc

B = 16384
D = 32
NC = 2
NS = 16
NW = NC * NS
BPW = B // NW
CHUNK = 128
NCHUNK = BPW // CHUNK
ROWS_PER_LINE = 128 // D


def _sc_body(uidx_hbm, iidx_hbm, uln_hbm, iln_hbm, ubf_hbm, ibf_hbm,
             partials_hbm, bias_hbm,
             uidx_v, iidx_v, ugid_v, igid_v, ugrp_v, igrp_v,
             ubv_v, ibv_v, acc_v, outb_v,
             sem_u, sem_i, sem_ub, sem_ib):
    wid = lax.axis_index("s") * NC + lax.axis_index("c")
    base = wid * BPW
    row0 = wid * NCHUNK

    pltpu.sync_copy(uidx_hbm.at[pl.ds(row0, NCHUNK)], uidx_v)
    pltpu.sync_copy(iidx_hbm.at[pl.ds(row0, NCHUNK)], iidx_v)

    bias_copies = []
    for j in range(NCHUNK):
        bias_copies.append(pltpu.async_copy(
            ubf_hbm.at[uidx_v.at[j]], ubv_v.at[pl.ds(j * CHUNK, CHUNK)],
            sem_ub))
        bias_copies.append(pltpu.async_copy(
            ibf_hbm.at[iidx_v.at[j]], ibv_v.at[pl.ds(j * CHUNK, CHUNK)],
            sem_ib))

    for j in range(NCHUNK):
        for k in range(CHUNK // 16):
            u = uidx_v[j, pl.ds(k * 16, 16)]
            i = iidx_v[j, pl.ds(k * 16, 16)]
            ugid_v[j, pl.ds(k * 16, 16)] = lax.shift_right_logical(u, 2)
            igid_v[j, pl.ds(k * 16, 16)] = lax.shift_right_logical(i, 2)

    lanes = lax.iota(jnp.int32, 16)
    acc = jnp.zeros((16,), jnp.float32)

    for j in range(NCHUNK):
        cu = pltpu.async_copy(uln_hbm.at[ugid_v.at[j]], ugrp_v, sem_u)
        ci = pltpu.async_copy(iln_hbm.at[igid_v.at[j]], igrp_v, sem_i)
        cu.wait()
        ci.wait()
        for k in range(CHUNK // 16):
            uoff = (uidx_v[j, pl.ds(k * 16, 16)] & 3) * D
            ioff = (iidx_v[j, pl.ds(k * 16, 16)] & 3) * D
            rows = lanes + k * 16
            for d in range(D):
                ud = plsc.load_gather(ugrp_v, [rows, uoff + d])
                vd = plsc.load_gather(igrp_v, [rows, ioff + d])
                acc = acc + ud * vd

    acc_v[...] = acc
    pltpu.sync_copy(acc_v, partials_hbm.at[wid])

    for c in bias_copies:
        c.wait()
    for j in range(BPW // 16):
        outb_v[pl.ds(j * 16, 16)] = (
            ubv_v[pl.ds(j * 16, 16)] + ibv_v[pl.ds(j * 16, 16)])
    pltpu.sync_copy(outb_v, bias_hbm.at[pl.ds(base, BPW)])


@functools.cache
def _make_sc_call():
    return pl.kernel(
        _sc_body,
        out_type=[
            jax.ShapeDtypeStruct((NW, 16), jnp.float32),
            jax.ShapeDtypeStruct((B,), jnp.float32),
        ],
        mesh=plsc.VectorSubcoreMesh(
            core_axis_name="c", subcore_axis_name="s"),
        compiler_params=pltpu.CompilerParams(
            use_tc_tiling_on_sc=False, needs_layout_passes=False),
        scratch_types=[
            pltpu.VMEM((NCHUNK, CHUNK), jnp.int32),
            pltpu.VMEM((NCHUNK, CHUNK), jnp.int32),
            pltpu.VMEM((NCHUNK, CHUNK), jnp.int32),
            pltpu.VMEM((NCHUNK, CHUNK), jnp.int32),
            pltpu.VMEM((CHUNK, 128), jnp.float32),
            pltpu.VMEM((CHUNK, 128), jnp.float32),
            pltpu.VMEM((BPW,), jnp.float32),
            pltpu.VMEM((BPW,), jnp.float32),
            pltpu.VMEM((16,), jnp.float32),
            pltpu.VMEM((BPW,), jnp.float32),
            pltpu.SemaphoreType.DMA,
            pltpu.SemaphoreType.DMA,
            pltpu.SemaphoreType.DMA,
            pltpu.SemaphoreType.DMA,
        ],
    )


def _tc_body(bias_ref, partials_ref, out_ref):
    total = jnp.sum(partials_ref[...])
    out_ref[...] = bias_ref[...] + total


_tc_call = pl.pallas_call(
    _tc_body,
    out_shape=jax.ShapeDtypeStruct((128, 128), jnp.float32),
)


def kernel(user_index, item_index, user_emb, item_emb, ub, ib):
    uidx2d = user_index.astype(jnp.int32).reshape(B // CHUNK, CHUNK)
    iidx2d = item_index.astype(jnp.int32).reshape(B // CHUNK, CHUNK)
    uln = user_emb.reshape(-1, 128)
    iln = item_emb.reshape(-1, 128)
    ubf = ub.reshape(-1)
    ibf = ib.reshape(-1)
    partials, bias = _make_sc_call()(uidx2d, iidx2d, uln, iln, ubf, ibf)
    out2d = _tc_call(bias.reshape(128, 128), partials)
    return out2d.reshape(B)
